# baseline (speedup 1.0000x reference)
import jax
import jax.numpy as jnp
from jax import lax
from jax.experimental import pallas as pl
from jax.experimental.pallas import tpu as pltpu
from jax.experimental.pallas import tpu_sc as plsc

N = 50000
D = 32
E = 800000
NLISTS = 8

NC = 2
NS = 16
B = 96
CH = 8
MAC = 33
NPROC = 2 * MAC
EPT = NPROC * CH * B
E_PAD = EPT * NS
CE = CH * B

ACC_N = 50176
RPT = ACC_N // NS

R = 1792
NBLK = ACC_N // R


def _sc_agg(feat_hbm, edges_hbm, zrows_hbm, zdeg_hbm, acc_out, deg_out,
            acc_sp, deg_sp, ebufA, ebufB, si0, si1, si2, si3,
            di0, di1, di2, di3, r0, r1, r2, r3, ones_v,
            gs0, gs1, gs2, gs3, ss0, ss1, ss2, ss3, dsem, isA, isB):
    c = lax.axis_index("c")
    s = lax.axis_index("s")
    rows = (r0, r1, r2, r3)
    sidx = (si0, si1, si2, si3)
    didx = (di0, di1, di2, di3)
    gsem = (gs0, gs1, gs2, gs3)
    ssem = (ss0, ss1, ss2, ss3)

    one16 = jnp.ones((16,), jnp.float32)
    for j in range(B // 16):
        ones_v[pl.ds(j * 16, 16)] = one16

    iota16 = lax.iota(jnp.int32, 16)
    zero16i = jnp.zeros((16,), jnp.int32)
    one16i = jnp.ones((16,), jnp.int32)

    row0 = s * RPT
    tbe = s * EPT

    for li in range(4):
        l = c * 4 + li
        if li < 2:
            off = jnp.where(c == 0, ACC_N, 0)
        else:
            off = jnp.where(c == 0, 0, ACC_N)

        def _deint(ebuf, brow, x, off=off):
            for g in range(B // 16):
                rr = brow * B + g * 16 + iota16
                dst16 = plsc.load_gather(ebuf, [rr, zero16i])
                src16 = plsc.load_gather(ebuf, [rr, one16i]) + off
                didx[x][pl.ds(g * 16, 16)] = dst16
                sidx[x][pl.ds(g * 16, 16)] = src16

        pltpu.sync_copy(zrows_hbm.at[pl.ds(row0, RPT), :],
                        acc_sp.at[pl.ds(row0, RPT), :])
        pltpu.sync_copy(zdeg_hbm.at[pl.ds(row0, RPT)],
                        deg_sp.at[pl.ds(row0, RPT)])
        plsc.subcore_barrier()

        pltpu.async_copy(edges_hbm.at[l, pl.ds(tbe, CE), :],
                         ebufA, isA).wait()
        _deint(ebufA, 0, 0)
        _deint(ebufA, 1, 1)
        pltpu.async_copy(feat_hbm.at[sidx[0]], rows[0], gsem[0])
        pltpu.async_copy(feat_hbm.at[sidx[1]], rows[1], gsem[1])

        def _macro(m, _):
            for u in range(16):
                cur = u % 4
                nx = (u + 2) % 4
                v = u + 2
                nbuf, nrow = (ebufA, v) if v < 8 else \
                    ((ebufB, v - 8) if v < 16 else (ebufA, v - 16))

                if u == 2:
                    pltpu.async_copy(
                        edges_hbm.at[l, pl.ds(tbe + CE * (2 * m + 1), CE), :],
                        ebufB, isB)
                if u == 5:
                    pltpu.make_async_copy(
                        edges_hbm.at[l, pl.ds(tbe + CE * (2 * m + 1), CE), :],
                        ebufB, isB).wait()
                if u == 10:
                    @pl.when(m < MAC - 1)
                    def _():
                        pltpu.async_copy(
                            edges_hbm.at[l, pl.ds(tbe + CE * (2 * m + 2),
                                                  CE), :],
                            ebufA, isA)
                if u == 13:
                    @pl.when(m < MAC - 1)
                    def _():
                        pltpu.make_async_copy(
                            edges_hbm.at[l, pl.ds(tbe + CE * (2 * m + 2),
                                                  CE), :],
                            ebufA, isA).wait()

                def _drain_sc():
                    pltpu.make_async_copy(rows[nx], acc_sp.at[didx[nx]],
                                          ssem[nx]).wait()

                def _drain_dg():
                    pltpu.make_async_copy(ones_v, deg_sp.at[didx[cur]],
                                          dsem).wait()

                def _issue_g():
                    _deint(nbuf, nrow, nx)
                    pltpu.async_copy(feat_hbm.at[sidx[nx]], rows[nx],
                                     gsem[nx])

                if u < 2:
                    @pl.when(m > 0)
                    def _():
                        _drain_sc()
                        _drain_dg()
                    _issue_g()
                elif u < 14:
                    _drain_sc()
                    _drain_dg()
                    _issue_g()
                else:
                    _drain_dg()

                    @pl.when(m < MAC - 1)
                    def _():
                        _drain_sc()
                        _issue_g()

                pltpu.make_async_copy(feat_hbm.at[sidx[cur]], rows[cur],
                                      gsem[cur]).wait()
                pltpu.async_copy(rows[cur], acc_sp.at[didx[cur]],
                                 ssem[cur], add=True)
                pltpu.async_copy(ones_v, deg_sp.at[didx[cur]],
                                 dsem, add=True)
            return _
        lax.fori_loop(0, MAC, _macro, None)

        for x in range(4):
            pltpu.make_async_copy(rows[x], acc_sp.at[didx[x]],
                                  ssem[x]).wait()
        for x in (2, 3):
            pltpu.make_async_copy(ones_v, deg_sp.at[didx[x]], dsem).wait()
        plsc.subcore_barrier()

        pltpu.sync_copy(acc_sp.at[pl.ds(row0, RPT), :],
                        acc_out.at[l, pl.ds(row0, RPT), :])
        pltpu.sync_copy(deg_sp.at[pl.ds(row0, RPT)],
                        deg_out.at[pl.ds(l * ACC_N + row0, RPT)])


_sc_agg_call = pl.kernel(
    _sc_agg,
    out_type=(jax.ShapeDtypeStruct((NLISTS, ACC_N, D), jnp.float32),
              jax.ShapeDtypeStruct((NLISTS * ACC_N,), jnp.float32)),
    mesh=plsc.VectorSubcoreMesh(core_axis_name="c", subcore_axis_name="s",
                                num_cores=NC, num_subcores=NS),
    compiler_params=pltpu.CompilerParams(use_tc_tiling_on_sc=False,
                                         needs_layout_passes=False,
                                         internal_scratch_in_bytes=131072),
    scratch_types=[
        pltpu.VMEM_SHARED((ACC_N, D), jnp.float32),
        pltpu.VMEM_SHARED((ACC_N,), jnp.float32),
        pltpu.VMEM((CE, 2), jnp.int32),
        pltpu.VMEM((CE, 2), jnp.int32),
    ] + [pltpu.VMEM((B,), jnp.int32)] * 8 + [
        pltpu.VMEM((B, D), jnp.float32),
        pltpu.VMEM((B, D), jnp.float32),
        pltpu.VMEM((B, D), jnp.float32),
        pltpu.VMEM((B, D), jnp.float32),
        pltpu.VMEM((B,), jnp.float32),
    ] + [pltpu.SemaphoreType.DMA] * 11,
)


def _tc_update(feat_ref, acc_ref, deg_ref, wagg_ref, bagg_ref,
               w1_ref, b1_ref, w2_ref, b2_ref, alpha_ref, out_ref):
    t = pl.program_id(0)
    deg8 = deg_ref[...]
    deg8 = jnp.where(deg8 == 0.0, 1.0, deg8)
    degs = jnp.where(t == 0, deg8[0:4], deg8[4:8])
    degb = lax.broadcast_in_dim(degs, (4, R, D), (0, 1))
    means = acc_ref[...] / degb
    dn = (((1,), (1,)), ((), ()))
    ms = [lax.dot_general(means[i], wagg_ref[i], dn,
                          preferred_element_type=jnp.float32)
          + bagg_ref[i][None, :] for i in range(4)]
    h = jnp.concatenate([feat_ref[...]] + ms, axis=1)
    u = lax.dot_general(h, w1_ref[...], dn,
                        preferred_element_type=jnp.float32) + b1_ref[...]
    a = alpha_ref[0, 0]
    u = jnp.where(u >= 0.0, u, a * u)
    out_ref[...] = lax.dot_general(u, w2_ref[...], dn,
                                   preferred_element_type=jnp.float32) \
        + b2_ref[...]


def _tc_call(feat_cat, acc, deg2, W_agg, b_agg, W_u1, b1, W_u2, b2, alpha):
    return pl.pallas_call(
        _tc_update,
        grid=(2, NBLK),
        in_specs=[
            pl.BlockSpec((R, D), lambda t, j: (t * NBLK + j, 0)),
            pl.BlockSpec((4, R, D), lambda t, j: (t, j, 0)),
            pl.BlockSpec((8, R), lambda t, j: (0, j)),
            pl.BlockSpec((4, D, D), lambda t, j: (t, 0, 0)),
            pl.BlockSpec((None, 4, D), lambda t, j: (t, 0, 0)),
            pl.BlockSpec((2 * D, 5 * D), lambda t, j: (0, 0)),
            pl.BlockSpec((1, 2 * D), lambda t, j: (0, 0)),
            pl.BlockSpec((D, 2 * D), lambda t, j: (0, 0)),
            pl.BlockSpec((1, D), lambda t, j: (0, 0)),
            pl.BlockSpec(memory_space=pltpu.SMEM),
        ],
        out_specs=pl.BlockSpec((None, R, D), lambda t, j: (t, j, 0)),
        out_shape=jax.ShapeDtypeStruct((2, ACC_N, D), jnp.float32),
    )(feat_cat, acc, deg2, W_agg, b_agg, W_u1, b1, W_u2, b2, alpha)


def kernel(feature_a, feature_b,
           edgelist_a_b_pos, edgelist_a_b_neg, edgelist_a_a_pos,
           edgelist_a_a_neg, edgelist_b_a_pos, edgelist_b_a_neg,
           edgelist_b_b_pos, edgelist_b_b_neg,
           W_agg, b_agg, W_u1, b_u1, prelu_a, W_u2, b_u2):
    zf = jnp.zeros((ACC_N - N, D), jnp.float32)
    feat_cat = jnp.concatenate([feature_a, zf, feature_b, zf], axis=0)

    edge_lists = (edgelist_a_b_pos, edgelist_a_b_neg, edgelist_a_a_pos,
                  edgelist_a_a_neg, edgelist_b_a_pos, edgelist_b_a_neg,
                  edgelist_b_b_pos, edgelist_b_b_neg)

    npad = E_PAD - E
    k = jnp.arange(npad, dtype=jnp.int32)
    pad2 = jnp.stack([N + k % (ACC_N - N), k % N], axis=1)
    edges_all = jnp.stack(
        [jnp.concatenate([e, pad2], axis=0) for e in edge_lists])

    zrows = jnp.zeros((ACC_N, D), jnp.float32)
    zdeg = jnp.zeros((ACC_N,), jnp.float32)

    acc, deg = _sc_agg_call(feat_cat, edges_all, zrows, zdeg)
    deg2 = deg.reshape(NLISTS, ACC_N)

    out = _tc_call(feat_cat, acc, deg2, W_agg, b_agg.reshape(2, 4, D),
                   W_u1, b_u1.reshape(1, 2 * D), W_u2,
                   b_u2.reshape(1, D), prelu_a.reshape(1, 1))
    return (out[0, :N], out[1, :N])


# speedup vs baseline: 7.3720x; 7.3720x over previous
import jax
import jax.numpy as jnp
from jax import lax
from jax.experimental import pallas as pl
from jax.experimental.pallas import tpu as pltpu
from jax.experimental.pallas import tpu_sc as plsc

N = 50000
D = 32
E = 800000
NLISTS = 8

NC = 2
NS = 16
B = 128
CH = 8
MAC = 25
NPROC = 2 * MAC
BPT = NPROC * CH
EPT = BPT * B
E_PAD = EPT * NS
NB = E_PAD // B

ACC_N = 50176
RPT = ACC_N // NS

R = 1792
NBLK = ACC_N // R


def _sc_agg(feat_hbm, src_hbm, dst_hbm, zrows_hbm, zdeg_hbm,
            acc_out, deg_out,
            acc_sp, deg_sp, srcA, dstA, srcB, dstB, r0, r1, r2, r3, ones_v,
            gs0, gs1, gs2, gs3, ss0, ss1, ss2, ss3, dsem, isA, isB):
    c = lax.axis_index("c")
    s = lax.axis_index("s")
    rows = (r0, r1, r2, r3)
    gsem = (gs0, gs1, gs2, gs3)
    ssem = (ss0, ss1, ss2, ss3)

    one16 = jnp.ones((16,), jnp.float32)
    for j in range(B // 16):
        ones_v[pl.ds(j * 16, 16)] = one16

    row0 = s * RPT
    tb = s * BPT

    for li in range(4):
        l = c * 4 + li

        def _load(csel, which, do_wait, l=l):
            sbuf, dbuf, sem = which
            ds_ = pl.ds(tb + CH * csel, CH)
            if do_wait:
                pltpu.make_async_copy(src_hbm.at[l, ds_], sbuf, sem).wait()
                pltpu.make_async_copy(dst_hbm.at[l, ds_], dbuf, sem).wait()
            else:
                pltpu.async_copy(src_hbm.at[l, ds_], sbuf, sem)
                pltpu.async_copy(dst_hbm.at[l, ds_], dbuf, sem)

        bufsA = (srcA, dstA, isA)
        bufsB = (srcB, dstB, isB)

        pltpu.sync_copy(zrows_hbm.at[pl.ds(row0, RPT), :],
                        acc_sp.at[pl.ds(row0, RPT), :])
        pltpu.sync_copy(zdeg_hbm.at[pl.ds(row0, RPT)],
                        deg_sp.at[pl.ds(row0, RPT)])
        plsc.subcore_barrier()

        _load(0, bufsA, False)
        _load(0, bufsA, True)
        pltpu.async_copy(feat_hbm.at[srcA.at[0]], rows[0], gsem[0])
        pltpu.async_copy(feat_hbm.at[srcA.at[1]], rows[1], gsem[1])

        def _macro(m, _):
            for u in range(16):
                cur = u % 4
                nx = (u + 2) % 4
                v = u + 2
                nbuf, nrow = (srcA, v) if v < 8 else \
                    ((srcB, v - 8) if v < 16 else (srcA, v - 16))
                cbuf, crow = (srcA, u) if u < 8 else (srcB, u - 8)
                cdbuf = dstA if u < 8 else dstB

                if u == 2:
                    _load(2 * m + 1, bufsB, False)
                if u == 5:
                    _load(2 * m + 1, bufsB, True)
                if u == 10:
                    @pl.when(m < MAC - 1)
                    def _():
                        _load(2 * m + 2, bufsA, False)
                if u == 13:
                    @pl.when(m < MAC - 1)
                    def _():
                        _load(2 * m + 2, bufsA, True)

                def _drain_sc(nx=nx, cdbuf=cdbuf, crow=crow):
                    pltpu.make_async_copy(rows[nx], acc_sp.at[cdbuf.at[crow]],
                                          ssem[nx]).wait()

                def _drain_dg(cdbuf=cdbuf, crow=crow):
                    pltpu.make_async_copy(ones_v, deg_sp.at[cdbuf.at[crow]],
                                          dsem).wait()

                def _issue_g(nx=nx, nbuf=nbuf, nrow=nrow):
                    pltpu.async_copy(feat_hbm.at[nbuf.at[nrow]], rows[nx],
                                     gsem[nx])

                if u < 2:
                    @pl.when(m > 0)
                    def _(d1=_drain_sc, d2=_drain_dg):
                        d1()
                        d2()
                    _issue_g()
                elif u < 14:
                    _drain_sc()
                    _drain_dg()
                    _issue_g()
                else:
                    _drain_dg()

                    @pl.when(m < MAC - 1)
                    def _(d1=_drain_sc, g=_issue_g):
                        d1()
                        g()

                pltpu.make_async_copy(feat_hbm.at[cbuf.at[crow]], rows[cur],
                                      gsem[cur]).wait()
                pltpu.async_copy(rows[cur], acc_sp.at[cdbuf.at[crow]],
                                 ssem[cur], add=True)
                pltpu.async_copy(ones_v, deg_sp.at[cdbuf.at[crow]],
                                 dsem, add=True)
            return _
        lax.fori_loop(0, MAC, _macro, None)

        for x in range(4):
            pltpu.make_async_copy(rows[x], acc_sp.at[dstB.at[x + 4]],
                                  ssem[x]).wait()
        for x in (2, 3):
            pltpu.make_async_copy(ones_v, deg_sp.at[dstB.at[x]], dsem).wait()
        plsc.subcore_barrier()

        pltpu.sync_copy(acc_sp.at[pl.ds(row0, RPT), :],
                        acc_out.at[l, pl.ds(row0, RPT), :])
        pltpu.sync_copy(deg_sp.at[pl.ds(row0, RPT)],
                        deg_out.at[pl.ds(l * ACC_N + row0, RPT)])


_sc_agg_call = pl.kernel(
    _sc_agg,
    out_type=(jax.ShapeDtypeStruct((NLISTS, ACC_N, D), jnp.float32),
              jax.ShapeDtypeStruct((NLISTS * ACC_N,), jnp.float32)),
    mesh=plsc.VectorSubcoreMesh(core_axis_name="c", subcore_axis_name="s",
                                num_cores=NC, num_subcores=NS),
    compiler_params=pltpu.CompilerParams(use_tc_tiling_on_sc=False),
    scratch_types=[
        pltpu.VMEM_SHARED((ACC_N, D), jnp.float32),
        pltpu.VMEM_SHARED((ACC_N,), jnp.float32),
        pltpu.VMEM((CH, B), jnp.int32),
        pltpu.VMEM((CH, B), jnp.int32),
        pltpu.VMEM((CH, B), jnp.int32),
        pltpu.VMEM((CH, B), jnp.int32),
        pltpu.VMEM((B, D), jnp.float32),
        pltpu.VMEM((B, D), jnp.float32),
        pltpu.VMEM((B, D), jnp.float32),
        pltpu.VMEM((B, D), jnp.float32),
        pltpu.VMEM((B,), jnp.float32),
    ] + [pltpu.SemaphoreType.DMA] * 11,
)


def _tc_update(feat_ref, acc_ref, deg_ref, wagg_ref, bagg_ref,
               w1_ref, b1_ref, w2_ref, b2_ref, alpha_ref, out_ref):
    t = pl.program_id(0)
    deg8 = deg_ref[...]
    deg8 = jnp.where(deg8 == 0.0, 1.0, deg8)
    degs = jnp.where(t == 0, deg8[0:4], deg8[4:8])
    degb = lax.broadcast_in_dim(degs, (4, R, D), (0, 1))
    means = acc_ref[...] / degb
    dn = (((1,), (1,)), ((), ()))
    ms = [lax.dot_general(means[i], wagg_ref[i], dn,
                          preferred_element_type=jnp.float32)
          + bagg_ref[i][None, :] for i in range(4)]
    h = jnp.concatenate([feat_ref[...]] + ms, axis=1)
    u = lax.dot_general(h, w1_ref[...], dn,
                        preferred_element_type=jnp.float32) + b1_ref[...]
    a = alpha_ref[0, 0]
    u = jnp.where(u >= 0.0, u, a * u)
    out_ref[...] = lax.dot_general(u, w2_ref[...], dn,
                                   preferred_element_type=jnp.float32) \
        + b2_ref[...]


def _tc_call(feat_cat, acc, deg2, W_agg, b_agg, W_u1, b1, W_u2, b2, alpha):
    return pl.pallas_call(
        _tc_update,
        grid=(2, NBLK),
        in_specs=[
            pl.BlockSpec((R, D), lambda t, j: (t * NBLK + j, 0)),
            pl.BlockSpec((4, R, D), lambda t, j: (t, j, 0)),
            pl.BlockSpec((8, R), lambda t, j: (0, j)),
            pl.BlockSpec((4, D, D), lambda t, j: (t, 0, 0)),
            pl.BlockSpec((None, 4, D), lambda t, j: (t, 0, 0)),
            pl.BlockSpec((2 * D, 5 * D), lambda t, j: (0, 0)),
            pl.BlockSpec((1, 2 * D), lambda t, j: (0, 0)),
            pl.BlockSpec((D, 2 * D), lambda t, j: (0, 0)),
            pl.BlockSpec((1, D), lambda t, j: (0, 0)),
            pl.BlockSpec(memory_space=pltpu.SMEM),
        ],
        out_specs=pl.BlockSpec((None, R, D), lambda t, j: (t, j, 0)),
        out_shape=jax.ShapeDtypeStruct((2, ACC_N, D), jnp.float32),
    )(feat_cat, acc, deg2, W_agg, b_agg, W_u1, b1, W_u2, b2, alpha)


def kernel(feature_a, feature_b,
           edgelist_a_b_pos, edgelist_a_b_neg, edgelist_a_a_pos,
           edgelist_a_a_neg, edgelist_b_a_pos, edgelist_b_a_neg,
           edgelist_b_b_pos, edgelist_b_b_neg,
           W_agg, b_agg, W_u1, b_u1, prelu_a, W_u2, b_u2):
    zf = jnp.zeros((ACC_N - N, D), jnp.float32)
    feat_cat = jnp.concatenate([feature_a, zf, feature_b, zf], axis=0)

    all_e = jnp.stack((edgelist_a_b_pos, edgelist_a_b_neg, edgelist_a_a_pos,
                       edgelist_a_a_neg, edgelist_b_a_pos, edgelist_b_a_neg,
                       edgelist_b_b_pos, edgelist_b_b_neg))
    offs = jnp.array([ACC_N, ACC_N, 0, 0, 0, 0, ACC_N, ACC_N], jnp.int32)

    npad = E_PAD - E
    k = jnp.arange(npad, dtype=jnp.int32)
    pad_dst = jnp.broadcast_to(N + k % (ACC_N - N), (NLISTS, npad))
    pad_src = jnp.broadcast_to(k % N, (NLISTS, npad))

    src_all = jnp.concatenate([all_e[:, :, 1] + offs[:, None], pad_src],
                              axis=1).reshape(NLISTS, NB, B)
    dst_all = jnp.concatenate([all_e[:, :, 0], pad_dst],
                              axis=1).reshape(NLISTS, NB, B)

    zrows = jnp.zeros((ACC_N, D), jnp.float32)
    zdeg = jnp.zeros((ACC_N,), jnp.float32)

    acc, deg = _sc_agg_call(feat_cat, src_all, dst_all, zrows, zdeg)
    deg2 = deg.reshape(NLISTS, ACC_N)

    out = _tc_call(feat_cat, acc, deg2, W_agg, b_agg.reshape(2, 4, D),
                   W_u1, b_u1.reshape(1, 2 * D), W_u2,
                   b_u2.reshape(1, D), prelu_a.reshape(1, 1))
    return (out[0, :N], out[1, :N])


# speedup vs baseline: 7.4782x; 1.0144x over previous
import jax
import jax.numpy as jnp
from jax import lax
from jax.experimental import pallas as pl
from jax.experimental.pallas import tpu as pltpu
from jax.experimental.pallas import tpu_sc as plsc

N = 50000
D = 32
E = 800000
NLISTS = 8

NC = 2
NS = 16
B = 128
CH = 8
MAC = 25
NPROC = 2 * MAC
BPT = NPROC * CH
EPT = BPT * B
E_PAD = EPT * NS
NB = E_PAD // B

ACC_N = 50176
RPT = ACC_N // NS

R = 3584
NBLK = ACC_N // R


def _sc_agg(feat_hbm, src_hbm, dst_hbm, zrows_hbm, zdeg_hbm,
            acc_out, deg_out,
            acc_sp, deg_sp, srcA, dstA, srcB, dstB, r0, r1, r2, r3, ones_v,
            gs0, gs1, gs2, gs3, ss0, ss1, ss2, ss3, dsem, isA, isB):
    c = lax.axis_index("c")
    s = lax.axis_index("s")
    rows = (r0, r1, r2, r3)
    gsem = (gs0, gs1, gs2, gs3)
    ssem = (ss0, ss1, ss2, ss3)

    one16 = jnp.ones((16,), jnp.float32)
    for j in range(B // 16):
        ones_v[pl.ds(j * 16, 16)] = one16

    row0 = s * RPT
    tb = s * BPT

    for li in range(4):
        l = c * 4 + li

        def _load(csel, which, do_wait, l=l):
            sbuf, dbuf, sem = which
            ds_ = pl.ds(tb + CH * csel, CH)
            if do_wait:
                pltpu.make_async_copy(src_hbm.at[l, ds_], sbuf, sem).wait()
                pltpu.make_async_copy(dst_hbm.at[l, ds_], dbuf, sem).wait()
            else:
                pltpu.async_copy(src_hbm.at[l, ds_], sbuf, sem)
                pltpu.async_copy(dst_hbm.at[l, ds_], dbuf, sem)

        bufsA = (srcA, dstA, isA)
        bufsB = (srcB, dstB, isB)

        pltpu.sync_copy(zrows_hbm.at[pl.ds(row0, RPT), :],
                        acc_sp.at[pl.ds(row0, RPT), :])
        pltpu.sync_copy(zdeg_hbm.at[pl.ds(row0, RPT)],
                        deg_sp.at[pl.ds(row0, RPT)])
        plsc.subcore_barrier()

        _load(0, bufsA, False)
        _load(0, bufsA, True)
        pltpu.async_copy(feat_hbm.at[srcA.at[0]], rows[0], gsem[0])
        pltpu.async_copy(feat_hbm.at[srcA.at[1]], rows[1], gsem[1])

        def _macro(m, _):
            for u in range(16):
                cur = u % 4
                nx = (u + 2) % 4
                v = u + 2
                nbuf, nrow = (srcA, v) if v < 8 else \
                    ((srcB, v - 8) if v < 16 else (srcA, v - 16))
                cbuf, crow = (srcA, u) if u < 8 else (srcB, u - 8)
                cdbuf = dstA if u < 8 else dstB

                if u == 2:
                    _load(2 * m + 1, bufsB, False)
                if u == 5:
                    _load(2 * m + 1, bufsB, True)
                if u == 10:
                    @pl.when(m < MAC - 1)
                    def _():
                        _load(2 * m + 2, bufsA, False)
                if u == 13:
                    @pl.when(m < MAC - 1)
                    def _():
                        _load(2 * m + 2, bufsA, True)

                def _drain_sc(nx=nx, cdbuf=cdbuf, crow=crow):
                    pltpu.make_async_copy(rows[nx], acc_sp.at[cdbuf.at[crow]],
                                          ssem[nx]).wait()

                def _drain_dg(cdbuf=cdbuf, crow=crow):
                    pltpu.make_async_copy(ones_v, deg_sp.at[cdbuf.at[crow]],
                                          dsem).wait()

                def _issue_g(nx=nx, nbuf=nbuf, nrow=nrow):
                    pltpu.async_copy(feat_hbm.at[nbuf.at[nrow]], rows[nx],
                                     gsem[nx])

                if u < 2:
                    @pl.when(m > 0)
                    def _(d1=_drain_sc, d2=_drain_dg):
                        d1()
                        d2()
                    _issue_g()
                elif u < 14:
                    _drain_sc()
                    _drain_dg()
                    _issue_g()
                else:
                    _drain_dg()

                    @pl.when(m < MAC - 1)
                    def _(d1=_drain_sc, g=_issue_g):
                        d1()
                        g()

                pltpu.make_async_copy(feat_hbm.at[cbuf.at[crow]], rows[cur],
                                      gsem[cur]).wait()
                pltpu.async_copy(rows[cur], acc_sp.at[cdbuf.at[crow]],
                                 ssem[cur], add=True)
                pltpu.async_copy(ones_v, deg_sp.at[cdbuf.at[crow]],
                                 dsem, add=True)
            return _
        lax.fori_loop(0, MAC, _macro, None)

        for x in range(4):
            pltpu.make_async_copy(rows[x], acc_sp.at[dstB.at[x + 4]],
                                  ssem[x]).wait()
        for x in (2, 3):
            pltpu.make_async_copy(ones_v, deg_sp.at[dstB.at[x]], dsem).wait()
        plsc.subcore_barrier()

        pltpu.sync_copy(acc_sp.at[pl.ds(row0, RPT), :],
                        acc_out.at[l, pl.ds(row0, RPT), :])
        pltpu.sync_copy(deg_sp.at[pl.ds(row0, RPT)],
                        deg_out.at[pl.ds(l * ACC_N + row0, RPT)])


_sc_agg_call = pl.kernel(
    _sc_agg,
    out_type=(jax.ShapeDtypeStruct((NLISTS, ACC_N, D), jnp.float32),
              jax.ShapeDtypeStruct((NLISTS * ACC_N,), jnp.float32)),
    mesh=plsc.VectorSubcoreMesh(core_axis_name="c", subcore_axis_name="s",
                                num_cores=NC, num_subcores=NS),
    compiler_params=pltpu.CompilerParams(use_tc_tiling_on_sc=False),
    scratch_types=[
        pltpu.VMEM_SHARED((ACC_N, D), jnp.float32),
        pltpu.VMEM_SHARED((ACC_N,), jnp.float32),
        pltpu.VMEM((CH, B), jnp.int32),
        pltpu.VMEM((CH, B), jnp.int32),
        pltpu.VMEM((CH, B), jnp.int32),
        pltpu.VMEM((CH, B), jnp.int32),
        pltpu.VMEM((B, D), jnp.float32),
        pltpu.VMEM((B, D), jnp.float32),
        pltpu.VMEM((B, D), jnp.float32),
        pltpu.VMEM((B, D), jnp.float32),
        pltpu.VMEM((B,), jnp.float32),
    ] + [pltpu.SemaphoreType.DMA] * 11,
)


def _tc_update(feat_ref, acc_ref, deg_ref, wagg_ref, bagg_ref,
               w1_ref, b1_ref, w2_ref, b2_ref, alpha_ref, out_ref):
    t = pl.program_id(0)
    deg8 = deg_ref[...]
    deg8 = jnp.where(deg8 == 0.0, 1.0, deg8)
    degs = jnp.where(t == 0, deg8[0:4], deg8[4:8])
    rdeg = 1.0 / degs
    degb = lax.broadcast_in_dim(rdeg, (4, R, D), (0, 1))
    means = acc_ref[...] * degb
    dn = (((1,), (1,)), ((), ()))
    ms = [lax.dot_general(means[i], wagg_ref[i], dn,
                          preferred_element_type=jnp.float32)
          + bagg_ref[i][None, :] for i in range(4)]
    h = jnp.concatenate([feat_ref[...]] + ms, axis=1)
    u = lax.dot_general(h, w1_ref[...], dn,
                        preferred_element_type=jnp.float32) + b1_ref[...]
    a = alpha_ref[0, 0]
    u = jnp.where(u >= 0.0, u, a * u)
    out_ref[...] = lax.dot_general(u, w2_ref[...], dn,
                                   preferred_element_type=jnp.float32) \
        + b2_ref[...]


def _tc_call(feat_cat, acc, deg2, W_agg, b_agg, W_u1, b1, W_u2, b2, alpha):
    return pl.pallas_call(
        _tc_update,
        grid=(2, NBLK),
        in_specs=[
            pl.BlockSpec((R, D), lambda t, j: (t * NBLK + j, 0)),
            pl.BlockSpec((4, R, D), lambda t, j: (t, j, 0)),
            pl.BlockSpec((8, R), lambda t, j: (0, j)),
            pl.BlockSpec((4, D, D), lambda t, j: (t, 0, 0)),
            pl.BlockSpec((None, 4, D), lambda t, j: (t, 0, 0)),
            pl.BlockSpec((2 * D, 5 * D), lambda t, j: (0, 0)),
            pl.BlockSpec((1, 2 * D), lambda t, j: (0, 0)),
            pl.BlockSpec((D, 2 * D), lambda t, j: (0, 0)),
            pl.BlockSpec((1, D), lambda t, j: (0, 0)),
            pl.BlockSpec(memory_space=pltpu.SMEM),
        ],
        out_specs=pl.BlockSpec((None, R, D), lambda t, j: (t, j, 0)),
        out_shape=jax.ShapeDtypeStruct((2, ACC_N, D), jnp.float32),
    )(feat_cat, acc, deg2, W_agg, b_agg, W_u1, b1, W_u2, b2, alpha)


def kernel(feature_a, feature_b,
           edgelist_a_b_pos, edgelist_a_b_neg, edgelist_a_a_pos,
           edgelist_a_a_neg, edgelist_b_a_pos, edgelist_b_a_neg,
           edgelist_b_b_pos, edgelist_b_b_neg,
           W_agg, b_agg, W_u1, b_u1, prelu_a, W_u2, b_u2):
    zf = jnp.zeros((ACC_N - N, D), jnp.float32)
    feat_cat = jnp.concatenate([feature_a, zf, feature_b, zf], axis=0)

    all_e = jnp.stack((edgelist_a_b_pos, edgelist_a_b_neg, edgelist_a_a_pos,
                       edgelist_a_a_neg, edgelist_b_a_pos, edgelist_b_a_neg,
                       edgelist_b_b_pos, edgelist_b_b_neg))
    offs = jnp.array([ACC_N, ACC_N, 0, 0, 0, 0, ACC_N, ACC_N], jnp.int32)

    npad = E_PAD - E
    k = jnp.arange(npad, dtype=jnp.int32)
    pad_dst = jnp.broadcast_to(N + k % (ACC_N - N), (NLISTS, npad))
    pad_src = jnp.broadcast_to(k % N, (NLISTS, npad))

    src_all = jnp.concatenate([all_e[:, :, 1] + offs[:, None], pad_src],
                              axis=1).reshape(NLISTS, NB, B)
    dst_all = jnp.concatenate([all_e[:, :, 0], pad_dst],
                              axis=1).reshape(NLISTS, NB, B)

    zrows = jnp.zeros((ACC_N, D), jnp.float32)
    zdeg = jnp.zeros((ACC_N,), jnp.float32)

    acc, deg = _sc_agg_call(feat_cat, src_all, dst_all, zrows, zdeg)
    deg2 = deg.reshape(NLISTS, ACC_N)

    out = _tc_call(feat_cat, acc, deg2, W_agg, b_agg.reshape(2, 4, D),
                   W_u1, b_u1.reshape(1, 2 * D), W_u2,
                   b_u2.reshape(1, D), prelu_a.reshape(1, 1))
    return (out[0, :N], out[1, :N])


# speedup vs baseline: 7.5705x; 1.0123x over previous
import jax
import jax.numpy as jnp
from jax import lax
from jax.experimental import pallas as pl
from jax.experimental.pallas import tpu as pltpu
from jax.experimental.pallas import tpu_sc as plsc

N = 50000
D = 32
E = 800000
NLISTS = 8

NC = 2
NS = 16
B = 128
CH = 8
MAC = 25
NPROC = 2 * MAC
BPT = NPROC * CH
EPT = BPT * B
E_PAD = EPT * NS
NB = E_PAD // B

ACC_N = 50176
RPT = ACC_N // NS

R = 3584
NBLK = ACC_N // R


def _sc_agg(feat_hbm, src_hbm, dst_hbm, zrows_hbm, zdeg_hbm,
            acc_out, deg_out,
            acc_sp, deg_sp, srcA, dstA, srcB, dstB, r0, r1, r2, r3, ones_v,
            gs0, gs1, gs2, gs3, ss0, ss1, ss2, ss3, dsem, isA, isB):
    c = lax.axis_index("c")
    s = lax.axis_index("s")
    rows = (r0, r1, r2, r3)
    gsem = (gs0, gs1, gs2, gs3)
    ssem = (ss0, ss1, ss2, ss3)

    one16 = jnp.ones((16,), jnp.float32)
    for j in range(B // 16):
        ones_v[pl.ds(j * 16, 16)] = one16

    row0 = s * RPT
    tb = s * BPT

    for li in range(2):
        l = c * 2 + li

        def _load(csel, which, do_wait, l=l):
            sbuf, dbuf, sem = which
            ds_ = pl.ds(tb + CH * csel, CH)
            if do_wait:
                pltpu.make_async_copy(src_hbm.at[l, ds_], sbuf, sem).wait()
                pltpu.make_async_copy(dst_hbm.at[l, ds_], dbuf, sem).wait()
            else:
                pltpu.async_copy(src_hbm.at[l, ds_], sbuf, sem)
                pltpu.async_copy(dst_hbm.at[l, ds_], dbuf, sem)

        bufsA = (srcA, dstA, isA)
        bufsB = (srcB, dstB, isB)

        pltpu.sync_copy(zrows_hbm.at[pl.ds(row0, RPT), :],
                        acc_sp.at[pl.ds(row0, RPT), :])
        pltpu.sync_copy(zdeg_hbm.at[pl.ds(row0, RPT)],
                        deg_sp.at[pl.ds(row0, RPT)])
        plsc.subcore_barrier()

        _load(0, bufsA, False)
        _load(0, bufsA, True)
        pltpu.async_copy(feat_hbm.at[srcA.at[0]], rows[0], gsem[0])
        pltpu.async_copy(feat_hbm.at[srcA.at[1]], rows[1], gsem[1])

        def _macro(m, _):
            for u in range(16):
                cur = u % 4
                nx = (u + 2) % 4
                v = u + 2
                nbuf, nrow = (srcA, v) if v < 8 else \
                    ((srcB, v - 8) if v < 16 else (srcA, v - 16))
                cbuf, crow = (srcA, u) if u < 8 else (srcB, u - 8)
                cdbuf = dstA if u < 8 else dstB

                if u == 2:
                    _load(2 * m + 1, bufsB, False)
                if u == 5:
                    _load(2 * m + 1, bufsB, True)
                if u == 10:
                    @pl.when(m < MAC - 1)
                    def _():
                        _load(2 * m + 2, bufsA, False)
                if u == 13:
                    @pl.when(m < MAC - 1)
                    def _():
                        _load(2 * m + 2, bufsA, True)

                def _drain_sc(nx=nx, cdbuf=cdbuf, crow=crow):
                    pltpu.make_async_copy(rows[nx], acc_sp.at[cdbuf.at[crow]],
                                          ssem[nx]).wait()

                def _drain_dg(cdbuf=cdbuf, crow=crow):
                    pltpu.make_async_copy(ones_v, deg_sp.at[cdbuf.at[crow]],
                                          dsem).wait()

                def _issue_g(nx=nx, nbuf=nbuf, nrow=nrow):
                    pltpu.async_copy(feat_hbm.at[nbuf.at[nrow]], rows[nx],
                                     gsem[nx])

                if u < 2:
                    @pl.when(m > 0)
                    def _(d1=_drain_sc, d2=_drain_dg):
                        d1()
                        d2()
                    _issue_g()
                elif u < 14:
                    _drain_sc()
                    _drain_dg()
                    _issue_g()
                else:
                    _drain_dg()

                    @pl.when(m < MAC - 1)
                    def _(d1=_drain_sc, g=_issue_g):
                        d1()
                        g()

                pltpu.make_async_copy(feat_hbm.at[cbuf.at[crow]], rows[cur],
                                      gsem[cur]).wait()
                pltpu.async_copy(rows[cur], acc_sp.at[cdbuf.at[crow]],
                                 ssem[cur], add=True)
                pltpu.async_copy(ones_v, deg_sp.at[cdbuf.at[crow]],
                                 dsem, add=True)
            return _
        lax.fori_loop(0, MAC, _macro, None)

        for x in range(4):
            pltpu.make_async_copy(rows[x], acc_sp.at[dstB.at[x + 4]],
                                  ssem[x]).wait()
        for x in (2, 3):
            pltpu.make_async_copy(ones_v, deg_sp.at[dstB.at[x]], dsem).wait()
        plsc.subcore_barrier()

        pltpu.sync_copy(acc_sp.at[pl.ds(row0, RPT), :],
                        acc_out.at[l, pl.ds(row0, RPT), :])
        pltpu.sync_copy(deg_sp.at[pl.ds(row0, RPT)],
                        deg_out.at[pl.ds(l * ACC_N + row0, RPT)])


_sc_agg_call = pl.kernel(
    _sc_agg,
    out_type=(jax.ShapeDtypeStruct((4, ACC_N, D), jnp.float32),
              jax.ShapeDtypeStruct((4 * ACC_N,), jnp.float32)),
    mesh=plsc.VectorSubcoreMesh(core_axis_name="c", subcore_axis_name="s",
                                num_cores=NC, num_subcores=NS),
    compiler_params=pltpu.CompilerParams(use_tc_tiling_on_sc=False),
    scratch_types=[
        pltpu.VMEM_SHARED((ACC_N, D), jnp.float32),
        pltpu.VMEM_SHARED((ACC_N,), jnp.float32),
        pltpu.VMEM((CH, B), jnp.int32),
        pltpu.VMEM((CH, B), jnp.int32),
        pltpu.VMEM((CH, B), jnp.int32),
        pltpu.VMEM((CH, B), jnp.int32),
        pltpu.VMEM((B, D), jnp.float32),
        pltpu.VMEM((B, D), jnp.float32),
        pltpu.VMEM((B, D), jnp.float32),
        pltpu.VMEM((B, D), jnp.float32),
        pltpu.VMEM((B,), jnp.float32),
    ] + [pltpu.SemaphoreType.DMA] * 11,
)


def _tc_update(feat_ref, acc_ref, deg_ref, wagg_ref, bagg_ref,
               w1_ref, b1_ref, w2_ref, b2_ref, alpha_ref, out_ref):
    degs = deg_ref[...]
    degs = jnp.where(degs == 0.0, 1.0, degs)
    rdeg = 1.0 / degs
    degb = lax.broadcast_in_dim(rdeg, (4, R, D), (0, 1))
    means = acc_ref[...] * degb
    dn = (((1,), (1,)), ((), ()))
    ms = [lax.dot_general(means[i], wagg_ref[i], dn,
                          preferred_element_type=jnp.float32)
          + bagg_ref[i][None, :] for i in range(4)]
    h = jnp.concatenate([feat_ref[...]] + ms, axis=1)
    u = lax.dot_general(h, w1_ref[...], dn,
                        preferred_element_type=jnp.float32) + b1_ref[...]
    a = alpha_ref[0, 0]
    u = jnp.where(u >= 0.0, u, a * u)
    out_ref[...] = lax.dot_general(u, w2_ref[...], dn,
                                   preferred_element_type=jnp.float32) \
        + b2_ref[...]


def _tc_call(side, feat_cat, acc, deg2, W_agg, b_agg,
             W_u1, b1, W_u2, b2, alpha):
    return pl.pallas_call(
        _tc_update,
        grid=(NBLK,),
        in_specs=[
            pl.BlockSpec((R, D), lambda j, side=side: (side * NBLK + j, 0)),
            pl.BlockSpec((4, R, D), lambda j: (0, j, 0)),
            pl.BlockSpec((4, R), lambda j: (0, j)),
            pl.BlockSpec((4, D, D), lambda j: (0, 0, 0)),
            pl.BlockSpec((4, D), lambda j: (0, 0)),
            pl.BlockSpec((2 * D, 5 * D), lambda j: (0, 0)),
            pl.BlockSpec((1, 2 * D), lambda j: (0, 0)),
            pl.BlockSpec((D, 2 * D), lambda j: (0, 0)),
            pl.BlockSpec((1, D), lambda j: (0, 0)),
            pl.BlockSpec(memory_space=pltpu.SMEM),
        ],
        out_specs=pl.BlockSpec((R, D), lambda j: (j, 0)),
        out_shape=jax.ShapeDtypeStruct((ACC_N, D), jnp.float32),
    )(feat_cat, acc, deg2, W_agg, b_agg, W_u1, b1, W_u2, b2, alpha)


def kernel(feature_a, feature_b,
           edgelist_a_b_pos, edgelist_a_b_neg, edgelist_a_a_pos,
           edgelist_a_a_neg, edgelist_b_a_pos, edgelist_b_a_neg,
           edgelist_b_b_pos, edgelist_b_b_neg,
           W_agg, b_agg, W_u1, b_u1, prelu_a, W_u2, b_u2):
    zf = jnp.zeros((ACC_N - N, D), jnp.float32)
    feat_cat = jnp.concatenate([feature_a, zf, feature_b, zf], axis=0)

    all_e = jnp.stack((edgelist_a_b_pos, edgelist_a_b_neg, edgelist_a_a_pos,
                       edgelist_a_a_neg, edgelist_b_a_pos, edgelist_b_a_neg,
                       edgelist_b_b_pos, edgelist_b_b_neg))
    offs = jnp.array([ACC_N, ACC_N, 0, 0, 0, 0, ACC_N, ACC_N], jnp.int32)

    npad = E_PAD - E
    k = jnp.arange(npad, dtype=jnp.int32)
    pad_dst = jnp.broadcast_to(N + k % (ACC_N - N), (NLISTS, npad))
    pad_src = jnp.broadcast_to(k % N, (NLISTS, npad))

    src_all = jnp.concatenate([all_e[:, :, 1] + offs[:, None], pad_src],
                              axis=1).reshape(NLISTS, NB, B)
    dst_all = jnp.concatenate([all_e[:, :, 0], pad_dst],
                              axis=1).reshape(NLISTS, NB, B)

    zrows = jnp.zeros((ACC_N, D), jnp.float32)
    zdeg = jnp.zeros((ACC_N,), jnp.float32)

    b_agg2 = b_agg.reshape(2, 4, D)
    b1 = b_u1.reshape(1, 2 * D)
    b2 = b_u2.reshape(1, D)
    alpha = prelu_a.reshape(1, 1)

    outs = []
    for side in (0, 1):
        acc, deg = _sc_agg_call(feat_cat, src_all[4 * side:4 * side + 4],
                                dst_all[4 * side:4 * side + 4], zrows, zdeg)
        outs.append(_tc_call(side, feat_cat, acc, deg.reshape(4, ACC_N),
                             W_agg[4 * side:4 * side + 4], b_agg2[side],
                             W_u1, b1, W_u2, b2, alpha))
    return (outs[0][:N], outs[1][:N])


# speedup vs baseline: 7.9223x; 1.0465x over previous
import jax
import jax.numpy as jnp
from jax import lax
from jax.experimental import pallas as pl
from jax.experimental.pallas import tpu as pltpu
from jax.experimental.pallas import tpu_sc as plsc

N = 50000
D = 32
E = 800000
NLISTS = 8

NC = 2
NS = 16
B = 128
CH = 8
MAC = 25
NPROC = 2 * MAC
BPT = NPROC * CH
EPT = BPT * B
E_PAD = EPT * NS
NB = E_PAD // B

ACC_N = 50176
RPT = ACC_N // NS

R = 3584
NBLK = ACC_N // R


def _sc_agg(feat_hbm, src_hbm, dst_hbm, zrows_hbm, zdeg_hbm,
            acc_out, deg_out,
            acc_sp, deg_sp, srcA, dstA, srcB, dstB, r0, r1, r2, r3, ones_v,
            gs0, gs1, gs2, gs3, ss0, ss1, ss2, ss3, dsem, isA, isB):
    c = lax.axis_index("c")
    s = lax.axis_index("s")
    rows = (r0, r1, r2, r3)
    gsem = (gs0, gs1, gs2, gs3)
    ssem = (ss0, ss1, ss2, ss3)

    one16 = jnp.ones((16,), jnp.float32)
    for j in range(B // 16):
        ones_v[pl.ds(j * 16, 16)] = one16

    row0 = s * RPT
    tb = s * BPT

    for li in range(2):
        l = c * 2 + li

        def _load(csel, which, do_wait, l=l):
            sbuf, dbuf, sem = which
            ds_ = pl.ds(tb + CH * csel, CH)
            if do_wait:
                pltpu.make_async_copy(src_hbm.at[l, ds_], sbuf, sem).wait()
                pltpu.make_async_copy(dst_hbm.at[l, ds_], dbuf, sem).wait()
            else:
                pltpu.async_copy(src_hbm.at[l, ds_], sbuf, sem)
                pltpu.async_copy(dst_hbm.at[l, ds_], dbuf, sem)

        bufsA = (srcA, dstA, isA)
        bufsB = (srcB, dstB, isB)

        pltpu.sync_copy(zrows_hbm.at[pl.ds(row0, RPT), :],
                        acc_sp.at[pl.ds(row0, RPT), :])
        pltpu.sync_copy(zdeg_hbm.at[pl.ds(row0, RPT)],
                        deg_sp.at[pl.ds(row0, RPT)])
        plsc.subcore_barrier()

        _load(0, bufsA, False)
        _load(0, bufsA, True)
        pltpu.async_copy(feat_hbm.at[srcA.at[0]], rows[0], gsem[0])
        pltpu.async_copy(feat_hbm.at[srcA.at[1]], rows[1], gsem[1])

        def _macro(m, _):
            for u in range(16):
                cur = u % 4
                nx = (u + 2) % 4
                v = u + 2
                nbuf, nrow = (srcA, v) if v < 8 else \
                    ((srcB, v - 8) if v < 16 else (srcA, v - 16))
                cbuf, crow = (srcA, u) if u < 8 else (srcB, u - 8)
                cdbuf = dstA if u < 8 else dstB

                if u == 2:
                    _load(2 * m + 1, bufsB, False)
                if u == 5:
                    _load(2 * m + 1, bufsB, True)
                if u == 10:
                    @pl.when(m < MAC - 1)
                    def _():
                        _load(2 * m + 2, bufsA, False)
                if u == 13:
                    @pl.when(m < MAC - 1)
                    def _():
                        _load(2 * m + 2, bufsA, True)

                def _drain_sc(nx=nx, cdbuf=cdbuf, crow=crow):
                    pltpu.make_async_copy(rows[nx], acc_sp.at[cdbuf.at[crow]],
                                          ssem[nx]).wait()

                def _drain_dg(cdbuf=cdbuf, crow=crow):
                    pltpu.make_async_copy(ones_v, deg_sp.at[cdbuf.at[crow]],
                                          dsem).wait()

                def _issue_g(nx=nx, nbuf=nbuf, nrow=nrow):
                    pltpu.async_copy(feat_hbm.at[nbuf.at[nrow]], rows[nx],
                                     gsem[nx])

                if u < 2:
                    @pl.when(m > 0)
                    def _(d1=_drain_sc, d2=_drain_dg):
                        d1()
                        d2()
                    _issue_g()
                elif u < 14:
                    _drain_sc()
                    _drain_dg()
                    _issue_g()
                else:
                    _drain_dg()

                    @pl.when(m < MAC - 1)
                    def _(d1=_drain_sc, g=_issue_g):
                        d1()
                        g()

                pltpu.make_async_copy(feat_hbm.at[cbuf.at[crow]], rows[cur],
                                      gsem[cur]).wait()
                pltpu.async_copy(rows[cur], acc_sp.at[cdbuf.at[crow]],
                                 ssem[cur], add=True)
                pltpu.async_copy(ones_v, deg_sp.at[cdbuf.at[crow]],
                                 dsem, add=True)
            return _
        lax.fori_loop(0, MAC, _macro, None)

        for x in range(4):
            pltpu.make_async_copy(rows[x], acc_sp.at[dstB.at[x + 4]],
                                  ssem[x]).wait()
        for x in (2, 3):
            pltpu.make_async_copy(ones_v, deg_sp.at[dstB.at[x]], dsem).wait()
        plsc.subcore_barrier()

        pltpu.sync_copy(acc_sp.at[pl.ds(row0, RPT), :],
                        acc_out.at[l, pl.ds(row0, RPT), :])
        pltpu.sync_copy(deg_sp.at[pl.ds(row0, RPT)],
                        deg_out.at[pl.ds(l * ACC_N + row0, RPT)])


_sc_agg_call = pl.kernel(
    _sc_agg,
    out_type=(jax.ShapeDtypeStruct((4, ACC_N, D), jnp.float32),
              jax.ShapeDtypeStruct((4 * ACC_N,), jnp.float32)),
    mesh=plsc.VectorSubcoreMesh(core_axis_name="c", subcore_axis_name="s",
                                num_cores=NC, num_subcores=NS),
    compiler_params=pltpu.CompilerParams(use_tc_tiling_on_sc=False),
    scratch_types=[
        pltpu.VMEM_SHARED((ACC_N, D), jnp.float32),
        pltpu.VMEM_SHARED((ACC_N,), jnp.float32),
        pltpu.VMEM((CH, B), jnp.int32),
        pltpu.VMEM((CH, B), jnp.int32),
        pltpu.VMEM((CH, B), jnp.int32),
        pltpu.VMEM((CH, B), jnp.int32),
        pltpu.VMEM((B, D), jnp.float32),
        pltpu.VMEM((B, D), jnp.float32),
        pltpu.VMEM((B, D), jnp.float32),
        pltpu.VMEM((B, D), jnp.float32),
        pltpu.VMEM((B,), jnp.float32),
    ] + [pltpu.SemaphoreType.DMA] * 11,
)


def _tc_update(feat_ref, acc_ref, deg_ref, wagg_ref, bagg_ref,
               w1_ref, b1_ref, w2_ref, b2_ref, alpha_ref, out_ref):
    degs = deg_ref[...]
    degs = jnp.where(degs == 0.0, 1.0, degs)
    rdeg = 1.0 / degs
    degb = lax.broadcast_in_dim(rdeg, (4, R, D), (0, 1))
    means = acc_ref[...] * degb
    dn = (((1,), (1,)), ((), ()))
    ms = [lax.dot_general(means[i], wagg_ref[i], dn,
                          preferred_element_type=jnp.float32)
          + bagg_ref[i][None, :] for i in range(4)]
    h = jnp.concatenate([feat_ref[...]] + ms, axis=1)
    u = lax.dot_general(h, w1_ref[...], dn,
                        preferred_element_type=jnp.float32) + b1_ref[...]
    a = alpha_ref[0, 0]
    u = jnp.where(u >= 0.0, u, a * u)
    out_ref[...] = lax.dot_general(u, w2_ref[...], dn,
                                   preferred_element_type=jnp.float32) \
        + b2_ref[...]


def _tc_call(side, feat_cat, acc, deg2, W_agg, b_agg,
             W_u1, b1, W_u2, b2, alpha):
    return pl.pallas_call(
        _tc_update,
        grid=(NBLK,),
        in_specs=[
            pl.BlockSpec((R, D), lambda j, side=side: (side * NBLK + j, 0)),
            pl.BlockSpec((4, R, D), lambda j: (0, j, 0)),
            pl.BlockSpec((4, R), lambda j: (0, j)),
            pl.BlockSpec((4, D, D), lambda j: (0, 0, 0)),
            pl.BlockSpec((4, D), lambda j: (0, 0)),
            pl.BlockSpec((2 * D, 5 * D), lambda j: (0, 0)),
            pl.BlockSpec((1, 2 * D), lambda j: (0, 0)),
            pl.BlockSpec((D, 2 * D), lambda j: (0, 0)),
            pl.BlockSpec((1, D), lambda j: (0, 0)),
            pl.BlockSpec(memory_space=pltpu.SMEM),
        ],
        out_specs=pl.BlockSpec((R, D), lambda j: (j, 0)),
        out_shape=jax.ShapeDtypeStruct((ACC_N, D), jnp.float32),
    )(feat_cat, acc, deg2, W_agg, b_agg, W_u1, b1, W_u2, b2, alpha)


def kernel(feature_a, feature_b,
           edgelist_a_b_pos, edgelist_a_b_neg, edgelist_a_a_pos,
           edgelist_a_a_neg, edgelist_b_a_pos, edgelist_b_a_neg,
           edgelist_b_b_pos, edgelist_b_b_neg,
           W_agg, b_agg, W_u1, b_u1, prelu_a, W_u2, b_u2):
    zf = jnp.zeros((ACC_N - N, D), jnp.float32)
    feat_cat = jnp.concatenate([feature_a, zf, feature_b, zf], axis=0)

    lists_by_side = ((edgelist_a_b_pos, edgelist_a_b_neg, edgelist_a_a_pos,
                      edgelist_a_a_neg),
                     (edgelist_b_a_pos, edgelist_b_a_neg, edgelist_b_b_pos,
                      edgelist_b_b_neg))
    offs_by_side = (jnp.array([ACC_N, ACC_N, 0, 0], jnp.int32),
                    jnp.array([0, 0, ACC_N, ACC_N], jnp.int32))

    npad = E_PAD - E
    k = jnp.arange(npad, dtype=jnp.int32)
    pad_dst = jnp.broadcast_to(N + k % (ACC_N - N), (4, npad))
    pad_src = jnp.broadcast_to(k % N, (4, npad))

    zrows = jnp.zeros((ACC_N, D), jnp.float32)
    zdeg = jnp.zeros((ACC_N,), jnp.float32)

    b_agg2 = b_agg.reshape(2, 4, D)
    b1 = b_u1.reshape(1, 2 * D)
    b2 = b_u2.reshape(1, D)
    alpha = prelu_a.reshape(1, 1)

    outs = []
    for side in (0, 1):
        all_e = jnp.stack(lists_by_side[side])
        offs = offs_by_side[side]
        src_s = jnp.concatenate([all_e[:, :, 1] + offs[:, None], pad_src],
                                axis=1).reshape(4, NB, B)
        dst_s = jnp.concatenate([all_e[:, :, 0], pad_dst],
                                axis=1).reshape(4, NB, B)
        acc, deg = _sc_agg_call(feat_cat, src_s, dst_s, zrows, zdeg)
        outs.append(_tc_call(side, feat_cat, acc, deg.reshape(4, ACC_N),
                             W_agg[4 * side:4 * side + 4], b_agg2[side],
                             W_u1, b1, W_u2, b2, alpha))
    return (outs[0][:N], outs[1][:N])


# speedup vs baseline: 7.9226x; 1.0000x over previous
import jax
import jax.numpy as jnp
from jax import lax
from jax.experimental import pallas as pl
from jax.experimental.pallas import tpu as pltpu
from jax.experimental.pallas import tpu_sc as plsc

N = 50000
D = 32
E = 800000
NLISTS = 8

NC = 2
NS = 16
B = 128
CH = 8
MAC = 25
NPROC = 2 * MAC
BPT = NPROC * CH
EPT = BPT * B
E_PAD = EPT * NS
NB = E_PAD // B

ACC_N = 50176
RPT = ACC_N // NS

R = 7168
NBLK = ACC_N // R


def _sc_agg(feat_hbm, src_hbm, dst_hbm, zrows_hbm, zdeg_hbm,
            acc_out, deg_out,
            acc_sp, deg_sp, srcA, dstA, srcB, dstB, r0, r1, r2, r3, ones_v,
            gs0, gs1, gs2, gs3, ss0, ss1, ss2, ss3, dsem, isA, isB):
    c = lax.axis_index("c")
    s = lax.axis_index("s")
    rows = (r0, r1, r2, r3)
    gsem = (gs0, gs1, gs2, gs3)
    ssem = (ss0, ss1, ss2, ss3)

    one16 = jnp.ones((16,), jnp.float32)
    for j in range(B // 16):
        ones_v[pl.ds(j * 16, 16)] = one16

    row0 = s * RPT
    tb = s * BPT

    for li in range(2):
        l = c * 2 + li

        def _load(csel, which, do_wait, l=l):
            sbuf, dbuf, sem = which
            ds_ = pl.ds(tb + CH * csel, CH)
            if do_wait:
                pltpu.make_async_copy(src_hbm.at[l, ds_], sbuf, sem).wait()
                pltpu.make_async_copy(dst_hbm.at[l, ds_], dbuf, sem).wait()
            else:
                pltpu.async_copy(src_hbm.at[l, ds_], sbuf, sem)
                pltpu.async_copy(dst_hbm.at[l, ds_], dbuf, sem)

        bufsA = (srcA, dstA, isA)
        bufsB = (srcB, dstB, isB)

        pltpu.sync_copy(zrows_hbm.at[pl.ds(row0, RPT), :],
                        acc_sp.at[pl.ds(row0, RPT), :])
        pltpu.sync_copy(zdeg_hbm.at[pl.ds(row0, RPT)],
                        deg_sp.at[pl.ds(row0, RPT)])
        plsc.subcore_barrier()

        _load(0, bufsA, False)
        _load(0, bufsA, True)
        pltpu.async_copy(feat_hbm.at[srcA.at[0]], rows[0], gsem[0])
        pltpu.async_copy(feat_hbm.at[srcA.at[1]], rows[1], gsem[1])

        def _macro(m, _):
            for u in range(16):
                cur = u % 4
                nx = (u + 2) % 4
                v = u + 2
                nbuf, nrow = (srcA, v) if v < 8 else \
                    ((srcB, v - 8) if v < 16 else (srcA, v - 16))
                cbuf, crow = (srcA, u) if u < 8 else (srcB, u - 8)
                cdbuf = dstA if u < 8 else dstB

                if u == 2:
                    _load(2 * m + 1, bufsB, False)
                if u == 5:
                    _load(2 * m + 1, bufsB, True)
                if u == 10:
                    @pl.when(m < MAC - 1)
                    def _():
                        _load(2 * m + 2, bufsA, False)
                if u == 13:
                    @pl.when(m < MAC - 1)
                    def _():
                        _load(2 * m + 2, bufsA, True)

                def _drain_sc(nx=nx, cdbuf=cdbuf, crow=crow):
                    pltpu.make_async_copy(rows[nx], acc_sp.at[cdbuf.at[crow]],
                                          ssem[nx]).wait()

                def _drain_dg(cdbuf=cdbuf, crow=crow):
                    pltpu.make_async_copy(ones_v, deg_sp.at[cdbuf.at[crow]],
                                          dsem).wait()

                def _issue_g(nx=nx, nbuf=nbuf, nrow=nrow):
                    pltpu.async_copy(feat_hbm.at[nbuf.at[nrow]], rows[nx],
                                     gsem[nx])

                if u < 2:
                    @pl.when(m > 0)
                    def _(d1=_drain_sc, d2=_drain_dg):
                        d1()
                        d2()
                    _issue_g()
                elif u < 14:
                    _drain_sc()
                    _drain_dg()
                    _issue_g()
                else:
                    _drain_dg()

                    @pl.when(m < MAC - 1)
                    def _(d1=_drain_sc, g=_issue_g):
                        d1()
                        g()

                pltpu.make_async_copy(feat_hbm.at[cbuf.at[crow]], rows[cur],
                                      gsem[cur]).wait()
                pltpu.async_copy(rows[cur], acc_sp.at[cdbuf.at[crow]],
                                 ssem[cur], add=True)
                pltpu.async_copy(ones_v, deg_sp.at[cdbuf.at[crow]],
                                 dsem, add=True)
            return _
        lax.fori_loop(0, MAC, _macro, None)

        for x in range(4):
            pltpu.make_async_copy(rows[x], acc_sp.at[dstB.at[x + 4]],
                                  ssem[x]).wait()
        for x in (2, 3):
            pltpu.make_async_copy(ones_v, deg_sp.at[dstB.at[x]], dsem).wait()
        plsc.subcore_barrier()

        pltpu.sync_copy(acc_sp.at[pl.ds(row0, RPT), :],
                        acc_out.at[l, pl.ds(row0, RPT), :])
        pltpu.sync_copy(deg_sp.at[pl.ds(row0, RPT)],
                        deg_out.at[pl.ds(l * ACC_N + row0, RPT)])


_sc_agg_call = pl.kernel(
    _sc_agg,
    out_type=(jax.ShapeDtypeStruct((4, ACC_N, D), jnp.float32),
              jax.ShapeDtypeStruct((4 * ACC_N,), jnp.float32)),
    mesh=plsc.VectorSubcoreMesh(core_axis_name="c", subcore_axis_name="s",
                                num_cores=NC, num_subcores=NS),
    compiler_params=pltpu.CompilerParams(use_tc_tiling_on_sc=False),
    scratch_types=[
        pltpu.VMEM_SHARED((ACC_N, D), jnp.float32),
        pltpu.VMEM_SHARED((ACC_N,), jnp.float32),
        pltpu.VMEM((CH, B), jnp.int32),
        pltpu.VMEM((CH, B), jnp.int32),
        pltpu.VMEM((CH, B), jnp.int32),
        pltpu.VMEM((CH, B), jnp.int32),
        pltpu.VMEM((B, D), jnp.float32),
        pltpu.VMEM((B, D), jnp.float32),
        pltpu.VMEM((B, D), jnp.float32),
        pltpu.VMEM((B, D), jnp.float32),
        pltpu.VMEM((B,), jnp.float32),
    ] + [pltpu.SemaphoreType.DMA] * 11,
)


def _tc_update(feat_ref, acc_ref, deg_ref, wagg_ref, bagg_ref,
               w1_ref, b1_ref, w2_ref, b2_ref, alpha_ref, out_ref):
    degs = deg_ref[...]
    degs = jnp.where(degs == 0.0, 1.0, degs)
    rdeg = 1.0 / degs
    degb = lax.broadcast_in_dim(rdeg, (4, R, D), (0, 1))
    means = acc_ref[...] * degb
    dn = (((1,), (1,)), ((), ()))
    ms = [lax.dot_general(means[i], wagg_ref[i], dn,
                          preferred_element_type=jnp.float32)
          + bagg_ref[i][None, :] for i in range(4)]
    h = jnp.concatenate([feat_ref[...]] + ms, axis=1)
    u = lax.dot_general(h, w1_ref[...], dn,
                        preferred_element_type=jnp.float32) + b1_ref[...]
    a = alpha_ref[0, 0]
    u = jnp.where(u >= 0.0, u, a * u)
    out_ref[...] = lax.dot_general(u, w2_ref[...], dn,
                                   preferred_element_type=jnp.float32) \
        + b2_ref[...]


def _tc_call(side, feat_cat, acc, deg2, W_agg, b_agg,
             W_u1, b1, W_u2, b2, alpha):
    return pl.pallas_call(
        _tc_update,
        grid=(NBLK,),
        in_specs=[
            pl.BlockSpec((R, D), lambda j, side=side: (side * NBLK + j, 0)),
            pl.BlockSpec((4, R, D), lambda j: (0, j, 0)),
            pl.BlockSpec((4, R), lambda j: (0, j)),
            pl.BlockSpec((4, D, D), lambda j: (0, 0, 0)),
            pl.BlockSpec((4, D), lambda j: (0, 0)),
            pl.BlockSpec((2 * D, 5 * D), lambda j: (0, 0)),
            pl.BlockSpec((1, 2 * D), lambda j: (0, 0)),
            pl.BlockSpec((D, 2 * D), lambda j: (0, 0)),
            pl.BlockSpec((1, D), lambda j: (0, 0)),
            pl.BlockSpec(memory_space=pltpu.SMEM),
        ],
        out_specs=pl.BlockSpec((R, D), lambda j: (j, 0)),
        out_shape=jax.ShapeDtypeStruct((ACC_N, D), jnp.float32),
    )(feat_cat, acc, deg2, W_agg, b_agg, W_u1, b1, W_u2, b2, alpha)


def kernel(feature_a, feature_b,
           edgelist_a_b_pos, edgelist_a_b_neg, edgelist_a_a_pos,
           edgelist_a_a_neg, edgelist_b_a_pos, edgelist_b_a_neg,
           edgelist_b_b_pos, edgelist_b_b_neg,
           W_agg, b_agg, W_u1, b_u1, prelu_a, W_u2, b_u2):
    zf = jnp.zeros((ACC_N - N, D), jnp.float32)
    feat_cat = jnp.concatenate([feature_a, zf, feature_b, zf], axis=0)

    lists_by_side = ((edgelist_a_b_pos, edgelist_a_b_neg, edgelist_a_a_pos,
                      edgelist_a_a_neg),
                     (edgelist_b_a_pos, edgelist_b_a_neg, edgelist_b_b_pos,
                      edgelist_b_b_neg))
    offs_by_side = (jnp.array([ACC_N, ACC_N, 0, 0], jnp.int32),
                    jnp.array([0, 0, ACC_N, ACC_N], jnp.int32))

    npad = E_PAD - E
    k = jnp.arange(npad, dtype=jnp.int32)
    pad_dst = jnp.broadcast_to(N + k % (ACC_N - N), (4, npad))
    pad_src = jnp.broadcast_to(k % N, (4, npad))

    zrows = jnp.zeros((ACC_N, D), jnp.float32)
    zdeg = jnp.zeros((ACC_N,), jnp.float32)

    b_agg2 = b_agg.reshape(2, 4, D)
    b1 = b_u1.reshape(1, 2 * D)
    b2 = b_u2.reshape(1, D)
    alpha = prelu_a.reshape(1, 1)

    outs = []
    for side in (0, 1):
        all_e = jnp.stack(lists_by_side[side])
        offs = offs_by_side[side]
        src_s = jnp.concatenate([all_e[:, :, 1] + offs[:, None], pad_src],
                                axis=1).reshape(4, NB, B)
        dst_s = jnp.concatenate([all_e[:, :, 0], pad_dst],
                                axis=1).reshape(4, NB, B)
        acc, deg = _sc_agg_call(feat_cat, src_s, dst_s, zrows, zdeg)
        outs.append(_tc_call(side, feat_cat, acc, deg.reshape(4, ACC_N),
                             W_agg[4 * side:4 * side + 4], b_agg2[side],
                             W_u1, b1, W_u2, b2, alpha))
    return (outs[0][:N], outs[1][:N])
